# baseline (speedup 1.0000x reference)
import functools
import math

import jax
import jax.numpy as jnp
from jax import lax
from jax.experimental import pallas as pl
from jax.experimental.pallas import tpu as pltpu
from jax.experimental.pallas import tpu_sc as plsc

D_MODEL = 1024
D_FF = 4096
N_EXP = 8
TOKENS = 4096
CAP = int(math.ceil(TOKENS * 1.6 / N_EXP))
CP = 896
NSLOT = N_EXP * CP
ZROW = NSLOT
YO_ROWS = (N_EXP + 1) * CP

_NC = 2
_NS = 16
_NW = _NC * _NS
_L = 16

_STRIPE = TOKENS // _L
_NITER = _STRIPE


def _sc_mesh():
    return plsc.VectorSubcoreMesh(
        core_axis_name="c", subcore_axis_name="s", num_cores=_NC, num_subcores=_NS
    )


def _router_body(x_ref, wr_ref, br_ref, t12_ref):
    l = jnp.dot(x_ref[...], wr_ref[...], preferred_element_type=jnp.float32)
    l = l + br_ref[...]
    col = lax.broadcasted_iota(jnp.int32, l.shape, 1)
    m1 = jnp.max(l, axis=1, keepdims=True)
    a1 = jnp.min(jnp.where(l == m1, col, 128), axis=1)
    l2 = jnp.where(col == a1[:, None], -jnp.inf, l)
    m2 = jnp.max(l2, axis=1, keepdims=True)
    a2 = jnp.min(jnp.where(l2 == m2, col, 128), axis=1)
    t12_ref[0:1, :] = a1.reshape(1, -1)
    t12_ref[1:2, :] = a2.reshape(1, -1)


def _router(xf, wr_p, br_p):
    return pl.pallas_call(
        _router_body,
        out_shape=jax.ShapeDtypeStruct((2, TOKENS), jnp.int32),
    )(xf, wr_p, br_p)


_DNUMS = lax.GatherDimensionNumbers(
    offset_dims=(), collapsed_slice_dims=(0,), start_index_map=(0,)
)


def _perm(x, idx):
    return lax.gather(
        x, idx[:, None], _DNUMS, (1,), mode=lax.GatherScatterMode.PROMISE_IN_BOUNDS
    )


def _excl_prefix(x):
    lane = lax.iota(jnp.int32, _L)
    s = x
    for k in (1, 2, 4, 8):
        sh = _perm(s, jnp.maximum(lane - k, 0))
        s = s + jnp.where(lane >= k, sh, 0)
    sh = _perm(s, jnp.maximum(lane - 1, 0))
    return jnp.where(lane >= 1, sh, 0)


def _dispatch_body(t12_hbm, fwd_hbm, invt_hbm,
                   t1_v, t2_v, wexp_v, invm_v, tokm_v, zb_v, cnt_v, sem):
    wid = lax.axis_index("s") * _NC + lax.axis_index("c")

    def _seg(k):
        return pl.ds(k * _L, _L)

    _ONE = jnp.full((_L,), 1, jnp.int32)
    _NIL = jnp.full((_L,), 0, jnp.int32)

    def _cnt(m):
        return jnp.where(m, _ONE, _NIL)

    @pl.when(wid == 0)
    def _():
        pltpu.sync_copy(t12_hbm.at[0], t1_v)
        pltpu.sync_copy(t12_hbm.at[1], t2_v)

        def czero(i, c):
            cnt_v[pl.ds(i * _L, _L)] = jnp.zeros((_L,), jnp.int32)
            return c

        lax.fori_loop(0, 48, czero, 0)

        def p1(i, c):
            acc = [_NIL] * N_EXP
            for u in range(8):
                t1 = t1_v[pl.ds((i * 8 + u) * _L, _L)]
                t2 = t2_v[pl.ds((i * 8 + u) * _L, _L)]
                for e in range(N_EXP):
                    m = (t1 == e) | (t2 == e)
                    acc[e] = acc[e] + _cnt(m)
            for e in range(N_EXP):
                cnt_v[_seg(16 + e)] = cnt_v[_seg(16 + e)] + acc[e]
            return c

        lax.fori_loop(0, _NITER // 8, p1, 0)
        for e in range(N_EXP):
            cnt_v[_seg(e)] = _excl_prefix(cnt_v[_seg(16 + e)])

        def p2(i, c):
            pos = [cnt_v[_seg(e)] + cnt_v[_seg(24 + e)] for e in range(N_EXP)]
            wacc = [_NIL] * N_EXP
            for u in range(4):
                t1 = t1_v[pl.ds((i * 4 + u) * _L, _L)]
                t2 = t2_v[pl.ds((i * 4 + u) * _L, _L)]
                wexp = jnp.full((_L,), -1, jnp.int32)
                for e in range(N_EXP):
                    m = (t1 == e) | (t2 == e)
                    kept = m & (pos[e] < CAP)
                    pos[e] = pos[e] + _cnt(m)
                    wexp = jnp.where(kept, jnp.full((_L,), e, jnp.int32), wexp)
                wexp_v[pl.ds((i * 4 + u) * _L, _L)] = wexp
                for e in range(N_EXP):
                    wacc[e] = wacc[e] + _cnt(wexp == e)
            for e in range(N_EXP):
                cnt_v[_seg(24 + e)] = pos[e] - cnt_v[_seg(e)]
                cnt_v[_seg(32 + e)] = cnt_v[_seg(32 + e)] + wacc[e]
            return c

        lax.fori_loop(0, _NITER // 4, p2, 0)
        for e in range(N_EXP):
            cnt_v[_seg(8 + e)] = _excl_prefix(cnt_v[_seg(32 + e)])

        def p3(i, c):
            slotb = [cnt_v[_seg(8 + e)] + cnt_v[_seg(40 + e)] + e * CP
                     for e in range(N_EXP)]
            for u in range(8):
                wexp = wexp_v[pl.ds((i * 8 + u) * _L, _L)]
                inv = jnp.full((_L,), ZROW, jnp.int32)
                for e in range(N_EXP):
                    mw = wexp == e
                    inv = jnp.where(mw, slotb[e], inv)
                    slotb[e] = slotb[e] + _cnt(mw)
                invm_v[i, pl.ds(u * _L, _L)] = inv
                tokm_v[i, pl.ds(u * _L, _L)] = (
                    lax.iota(jnp.int32, _L) * _STRIPE + i * 8 + u
                )
            for e in range(N_EXP):
                cnt_v[_seg(40 + e)] = slotb[e] - cnt_v[_seg(8 + e)] - e * CP
            return c

        lax.fori_loop(0, _NITER // 8, p3, 0)

        pltpu.sync_copy(invm_v, invt_hbm)

        def memset(i, c):
            zb_v[pl.ds(i * _L, _L)] = jnp.zeros((_L,), jnp.int32)
            return c

        lax.fori_loop(0, (NSLOT + 8) // _L, memset, 0)
        pltpu.sync_copy(zb_v, fwd_hbm)

        def scat(j, c):
            pltpu.async_copy(tokm_v.at[j], fwd_hbm.at[invm_v.at[j]], sem)
            return c

        lax.fori_loop(0, TOKENS // 128, scat, 0)

        def drain(j, c):
            pltpu.make_async_copy(
                tokm_v.at[j], fwd_hbm.at[invm_v.at[j]], sem
            ).wait()
            return c

        lax.fori_loop(0, TOKENS // 128, drain, 0)


def _dispatch(t12_t):
    return pl.kernel(
        _dispatch_body,
        out_type=(
            jax.ShapeDtypeStruct((NSLOT + 8,), jnp.int32),
            jax.ShapeDtypeStruct((TOKENS // 128, 128), jnp.int32),
        ),
        mesh=_sc_mesh(),
        scratch_types=[
            pltpu.VMEM((TOKENS,), jnp.int32),
            pltpu.VMEM((TOKENS,), jnp.int32),
            pltpu.VMEM((TOKENS,), jnp.int32),
            pltpu.VMEM((TOKENS // 128, 128), jnp.int32),
            pltpu.VMEM((TOKENS // 128, 128), jnp.int32),
            pltpu.VMEM((NSLOT + 8,), jnp.int32),
            pltpu.VMEM((48 * _L,), jnp.int32),
            pltpu.SemaphoreType.DMA,
        ],
    )(t12_t)


_G_Q = NSLOT // 4
_G_CHUNK = _G_Q // _NW


def _gather_body(q, x_hbm, fwd_hbm, xi_hbm, idx_v, rows_v, sem):
    wid = lax.axis_index("s") * _NC + lax.axis_index("c")
    off = wid * _G_CHUNK
    pltpu.sync_copy(fwd_hbm.at[pl.ds(q * _G_Q + off, _G_CHUNK)], idx_v)
    pltpu.async_copy(x_hbm.at[idx_v], rows_v, sem).wait()
    pltpu.sync_copy(rows_v, xi_hbm.at[pl.ds(off, _G_CHUNK)])


def _gather_q(xf, fwd, q):
    return pl.kernel(
        functools.partial(_gather_body, q),
        out_type=jax.ShapeDtypeStruct((_G_Q, D_MODEL), jnp.float32),
        mesh=_sc_mesh(),
        scratch_types=[
            pltpu.VMEM((_G_CHUNK,), jnp.int32),
            pltpu.VMEM((_G_CHUNK, D_MODEL), jnp.float32),
            pltpu.SemaphoreType.DMA,
        ],
        name=f"moe_gather_{q}",
    )(xf, fwd)


_FB = 512


def _ffn_body(yo_in_ref, xi_ref, w1_ref, b1_ref, w2_ref, b2_ref, out_ref):
    f = pl.program_id(1)
    xi = xi_ref[...].astype(jnp.bfloat16)
    w1 = w1_ref[0].astype(jnp.bfloat16)
    h = jnp.dot(xi, w1, preferred_element_type=jnp.float32) + b1_ref[0]
    h = 0.5 * h * (1.0 + lax.erf(h * (1.0 / math.sqrt(2.0))))
    w2 = w2_ref[0].astype(jnp.bfloat16)
    y = jnp.dot(h.astype(jnp.bfloat16), w2, preferred_element_type=jnp.float32)

    @pl.when(f == 0)
    def _():
        out_ref[...] = y + b2_ref[0]

    @pl.when(f > 0)
    def _():
        out_ref[...] = out_ref[...] + y


def _ffn_q(k, yo_in, xi_q, W1, B1r, W2, B2r):
    nf = D_FF // _FB
    return pl.pallas_call(
        _ffn_body,
        grid=(2, nf),
        in_specs=[
            pl.BlockSpec(memory_space=pl.ANY),
            pl.BlockSpec((CP, D_MODEL), lambda e, f: (e, 0)),
            pl.BlockSpec((1, D_MODEL, _FB), lambda e, f: (2 * k + e, 0, f)),
            pl.BlockSpec((1, 1, _FB), lambda e, f: (2 * k + e, 0, f)),
            pl.BlockSpec((1, _FB, D_MODEL), lambda e, f: (2 * k + e, f, 0)),
            pl.BlockSpec((1, 1, D_MODEL), lambda e, f: (2 * k + e, 0, 0)),
        ],
        out_specs=pl.BlockSpec((CP, D_MODEL), lambda e, f: (2 * k + e, 0)),
        out_shape=jax.ShapeDtypeStruct((YO_ROWS, D_MODEL), jnp.float32),
        input_output_aliases={0: 0},
        name=f"moe_ffn_{k}",
    )(yo_in, xi_q, W1, B1r, W2, B2r)


_O_HALF = TOKENS // 2
_O_CHUNK = _O_HALF // _NW


def _outgather_body(half, yo_hbm, inv_hbm, out_hbm, idx_v, rows_v, sem):
    wid = lax.axis_index("s") * _NC + lax.axis_index("c")
    off = wid * _O_CHUNK
    pltpu.sync_copy(inv_hbm.at[pl.ds(half * _O_HALF + off, _O_CHUNK)], idx_v)
    pltpu.async_copy(yo_hbm.at[idx_v], rows_v, sem).wait()
    pltpu.sync_copy(rows_v, out_hbm.at[pl.ds(off, _O_CHUNK)])


def _outgather_half(yo, inv, half):
    return pl.kernel(
        functools.partial(_outgather_body, half),
        out_type=jax.ShapeDtypeStruct((_O_HALF, D_MODEL), jnp.float32),
        mesh=_sc_mesh(),
        scratch_types=[
            pltpu.VMEM((_O_CHUNK,), jnp.int32),
            pltpu.VMEM((_O_CHUNK, D_MODEL), jnp.float32),
            pltpu.SemaphoreType.DMA,
        ],
        name=f"moe_outgather_{half}",
    )(yo, inv)


def kernel(x, Wr, br, W1, B1, W2, B2):
    B, S, D = x.shape
    xf = x.reshape(-1, D)
    wr_p = jnp.pad(Wr, ((0, 0), (0, 128 - N_EXP)))
    br_p = jnp.concatenate([br, jnp.full((128 - N_EXP,), -1e30, jnp.float32)])
    t12 = _router(xf, wr_p, br_p.reshape(1, 128))
    t12_t = t12.reshape(2, _L, _STRIPE).transpose(0, 2, 1).reshape(2, TOKENS)
    fwd, invt = _dispatch(t12_t)
    inv = invt.reshape(_STRIPE, _L).transpose(1, 0).reshape(TOKENS)
    B1r = B1.reshape(N_EXP, 1, D_FF)
    B2r = B2.reshape(N_EXP, 1, D_MODEL)
    yo = jnp.zeros((YO_ROWS, D_MODEL), jnp.float32)
    for k in range(4):
        xi_k = _gather_q(xf, fwd, k)
        yo = _ffn_q(k, yo, xi_k, W1, B1r, W2, B2r)
    o0 = _outgather_half(yo, inv, 0)
    o1 = _outgather_half(yo, inv, 1)
    out = jnp.concatenate([o0, o1], axis=0)
    return out.reshape(B, S, D)


# speedup vs baseline: 1.7728x; 1.7728x over previous
import functools
import math

import jax
import jax.numpy as jnp
from jax import lax
from jax.experimental import pallas as pl
from jax.experimental.pallas import tpu as pltpu
from jax.experimental.pallas import tpu_sc as plsc

D_MODEL = 1024
D_FF = 4096
N_EXP = 8
TOKENS = 4096
CAP = int(math.ceil(TOKENS * 1.6 / N_EXP))
CP = 896
NSLOT = N_EXP * CP
ZROW = NSLOT
YO_ROWS = (N_EXP + 1) * CP

_NC = 2
_NS = 16
_NW = _NC * _NS
_L = 16

_STRIPE = TOKENS // _L
_NITER = _STRIPE


def _sc_mesh():
    return plsc.VectorSubcoreMesh(
        core_axis_name="c", subcore_axis_name="s", num_cores=_NC, num_subcores=_NS
    )


def _router_body(x_ref, wr_ref, br_ref, t12_ref):
    l = jnp.dot(x_ref[...], wr_ref[...], preferred_element_type=jnp.float32)
    l = l + br_ref[...]
    col = lax.broadcasted_iota(jnp.int32, l.shape, 1)
    m1 = jnp.max(l, axis=1, keepdims=True)
    a1 = jnp.min(jnp.where(l == m1, col, 128), axis=1)
    l2 = jnp.where(col == a1[:, None], -jnp.inf, l)
    m2 = jnp.max(l2, axis=1, keepdims=True)
    a2 = jnp.min(jnp.where(l2 == m2, col, 128), axis=1)
    t12_ref[0:1, :] = a1.reshape(1, -1)
    t12_ref[1:2, :] = a2.reshape(1, -1)


def _router(xf, wr_p, br_p):
    return pl.pallas_call(
        _router_body,
        out_shape=jax.ShapeDtypeStruct((2, TOKENS), jnp.int32),
    )(xf, wr_p, br_p)


_DNUMS = lax.GatherDimensionNumbers(
    offset_dims=(), collapsed_slice_dims=(0,), start_index_map=(0,)
)


def _perm(x, idx):
    return lax.gather(
        x, idx[:, None], _DNUMS, (1,), mode=lax.GatherScatterMode.PROMISE_IN_BOUNDS
    )


def _excl_prefix(x):
    lane = lax.iota(jnp.int32, _L)
    s = x
    for k in (1, 2, 4, 8):
        sh = _perm(s, jnp.maximum(lane - k, 0))
        s = s + jnp.where(lane >= k, sh, 0)
    sh = _perm(s, jnp.maximum(lane - 1, 0))
    return jnp.where(lane >= 1, sh, 0)


def _dispatch_body(t12_hbm, fwd_hbm, invt_hbm,
                   t1_v, t2_v, wexp_v, invm_v, tokm_v, zb_v, cnt_v,
                   sh_v, inv2_v, tok2_v, sem):
    wid = lax.axis_index("s") * _NC + lax.axis_index("c")

    def _seg(k):
        return pl.ds(k * _L, _L)

    _ONE = jnp.full((_L,), 1, jnp.int32)
    _NIL = jnp.full((_L,), 0, jnp.int32)

    def _cnt(m):
        return jnp.where(m, _ONE, _NIL)

    @pl.when(wid == 0)
    def _():
        pltpu.sync_copy(t12_hbm.at[0], t1_v)
        pltpu.sync_copy(t12_hbm.at[1], t2_v)

        def czero(i, c):
            cnt_v[pl.ds(i * _L, _L)] = jnp.zeros((_L,), jnp.int32)
            return c

        lax.fori_loop(0, 48, czero, 0)

        def p1(i, c):
            acc = [_NIL] * N_EXP
            for u in range(8):
                t1 = t1_v[pl.ds((i * 8 + u) * _L, _L)]
                t2 = t2_v[pl.ds((i * 8 + u) * _L, _L)]
                for e in range(N_EXP):
                    m = (t1 == e) | (t2 == e)
                    acc[e] = acc[e] + _cnt(m)
            for e in range(N_EXP):
                cnt_v[_seg(16 + e)] = cnt_v[_seg(16 + e)] + acc[e]
            return c

        lax.fori_loop(0, _NITER // 8, p1, 0)
        for e in range(N_EXP):
            cnt_v[_seg(e)] = _excl_prefix(cnt_v[_seg(16 + e)])

        def p2(i, c):
            pos = [cnt_v[_seg(e)] + cnt_v[_seg(24 + e)] for e in range(N_EXP)]
            wacc = [_NIL] * N_EXP
            for u in range(4):
                t1 = t1_v[pl.ds((i * 4 + u) * _L, _L)]
                t2 = t2_v[pl.ds((i * 4 + u) * _L, _L)]
                wexp = jnp.full((_L,), -1, jnp.int32)
                for e in range(N_EXP):
                    m = (t1 == e) | (t2 == e)
                    kept = m & (pos[e] < CAP)
                    pos[e] = pos[e] + _cnt(m)
                    wexp = jnp.where(kept, jnp.full((_L,), e, jnp.int32), wexp)
                wexp_v[pl.ds((i * 4 + u) * _L, _L)] = wexp
                for e in range(N_EXP):
                    wacc[e] = wacc[e] + _cnt(wexp == e)
            for e in range(N_EXP):
                cnt_v[_seg(24 + e)] = pos[e] - cnt_v[_seg(e)]
                cnt_v[_seg(32 + e)] = cnt_v[_seg(32 + e)] + wacc[e]
            return c

        lax.fori_loop(0, _NITER // 4, p2, 0)
        for e in range(N_EXP):
            cnt_v[_seg(8 + e)] = _excl_prefix(cnt_v[_seg(32 + e)])

        def p3(i, c):
            slotb = [cnt_v[_seg(8 + e)] + cnt_v[_seg(40 + e)] + e * CP
                     for e in range(N_EXP)]
            for u in range(8):
                wexp = wexp_v[pl.ds((i * 8 + u) * _L, _L)]
                tok = lax.iota(jnp.int32, _L) * _STRIPE + i * 8 + u
                inv = ZROW + (tok & 511)
                for e in range(N_EXP):
                    mw = wexp == e
                    inv = jnp.where(mw, slotb[e], inv)
                    slotb[e] = slotb[e] + _cnt(mw)
                invm_v[i, pl.ds(u * _L, _L)] = inv
                tokm_v[i, pl.ds(u * _L, _L)] = tok
            for e in range(N_EXP):
                cnt_v[_seg(40 + e)] = slotb[e] - cnt_v[_seg(8 + e)] - e * CP
            return c

        lax.fori_loop(0, _NITER // 8, p3, 0)

        pltpu.sync_copy(invm_v, invt_hbm)
        pltpu.sync_copy(invm_v, sh_v.at[pl.ds(0, 32)])
        pltpu.sync_copy(tokm_v, sh_v.at[pl.ds(32, 32)])

    cid = lax.axis_index("c")
    sid = lax.axis_index("s")

    @pl.when(cid == 0)
    def _():
        base = sid * _ZCH

        def memset(i, c):
            zb_v[pl.ds(i * _L, _L)] = (
                (lax.iota(jnp.int32, _L) + base + i * _L) & (TOKENS - 1)
            )
            return c

        lax.fori_loop(0, _ZN // _L, memset, 0)

        @pl.when(sid < _NS - 1)
        def _():
            pltpu.sync_copy(zb_v.at[pl.ds(0, _ZCH)], fwd_hbm.at[pl.ds(base, _ZCH)])

        @pl.when(sid == _NS - 1)
        def _():
            pltpu.sync_copy(zb_v, fwd_hbm.at[pl.ds(base, _ZN)])

    plsc.subcore_barrier()

    @pl.when(cid == 0)
    def _():
        pltpu.sync_copy(sh_v.at[pl.ds(2 * sid, 2)], inv2_v)
        pltpu.sync_copy(sh_v.at[pl.ds(32 + 2 * sid, 2)], tok2_v)
        for j in range(2):
            pltpu.async_copy(tok2_v.at[j], fwd_hbm.at[inv2_v.at[j]], sem)
        for j in range(2):
            pltpu.make_async_copy(
                tok2_v.at[j], fwd_hbm.at[inv2_v.at[j]], sem
            ).wait()


_FWD_ROWS = NSLOT + 512
_ZCH = _FWD_ROWS // _NS // 8 * 8
_ZN = _FWD_ROWS - (_NS - 1) * _ZCH


def _dispatch(t12_t):
    return pl.kernel(
        _dispatch_body,
        out_type=(
            jax.ShapeDtypeStruct((_FWD_ROWS,), jnp.int32),
            jax.ShapeDtypeStruct((TOKENS // 128, 128), jnp.int32),
        ),
        mesh=_sc_mesh(),
        scratch_types=[
            pltpu.VMEM((TOKENS,), jnp.int32),
            pltpu.VMEM((TOKENS,), jnp.int32),
            pltpu.VMEM((TOKENS,), jnp.int32),
            pltpu.VMEM((TOKENS // 128, 128), jnp.int32),
            pltpu.VMEM((TOKENS // 128, 128), jnp.int32),
            pltpu.VMEM((_ZN,), jnp.int32),
            pltpu.VMEM((48 * _L,), jnp.int32),
            pltpu.VMEM_SHARED((64, 128), jnp.int32),
            pltpu.VMEM((2, 128), jnp.int32),
            pltpu.VMEM((2, 128), jnp.int32),
            pltpu.SemaphoreType.DMA,
        ],
    )(t12_t)


_G_Q = NSLOT // 4
_G_CHUNK = _G_Q // _NW


def _gather_body(q, x_hbm, fwd_hbm, xi_hbm, idx_v, rows_v, sem):
    wid = lax.axis_index("s") * _NC + lax.axis_index("c")
    off = wid * _G_CHUNK
    pltpu.sync_copy(fwd_hbm.at[pl.ds(q * _G_Q + off, _G_CHUNK)], idx_v)
    pltpu.async_copy(x_hbm.at[idx_v], rows_v, sem).wait()
    pltpu.sync_copy(rows_v, xi_hbm.at[pl.ds(off, _G_CHUNK)])


def _gather_q(xf, fwd, q):
    return pl.kernel(
        functools.partial(_gather_body, q),
        out_type=jax.ShapeDtypeStruct((_G_Q, D_MODEL), jnp.float32),
        mesh=_sc_mesh(),
        scratch_types=[
            pltpu.VMEM((_G_CHUNK,), jnp.int32),
            pltpu.VMEM((_G_CHUNK, D_MODEL), jnp.float32),
            pltpu.SemaphoreType.DMA,
        ],
        name=f"moe_gather_{q}",
    )(xf, fwd)


_FB = 512


def _ffn_body(yo_in_ref, xi_ref, w1_ref, b1_ref, w2_ref, b2_ref, out_ref):
    f = pl.program_id(1)
    xi = xi_ref[...].astype(jnp.bfloat16)
    w1 = w1_ref[0].astype(jnp.bfloat16)
    h = jnp.dot(xi, w1, preferred_element_type=jnp.float32) + b1_ref[0]
    h = 0.5 * h * (1.0 + lax.erf(h * (1.0 / math.sqrt(2.0))))
    w2 = w2_ref[0].astype(jnp.bfloat16)
    y = jnp.dot(h.astype(jnp.bfloat16), w2, preferred_element_type=jnp.float32)

    @pl.when(f == 0)
    def _():
        out_ref[...] = y + b2_ref[0]

    @pl.when(f > 0)
    def _():
        out_ref[...] = out_ref[...] + y


def _ffn_q(k, yo_in, xi_q, W1, B1r, W2, B2r):
    nf = D_FF // _FB
    return pl.pallas_call(
        _ffn_body,
        grid=(2, nf),
        in_specs=[
            pl.BlockSpec(memory_space=pl.ANY),
            pl.BlockSpec((CP, D_MODEL), lambda e, f: (e, 0)),
            pl.BlockSpec((1, D_MODEL, _FB), lambda e, f: (2 * k + e, 0, f)),
            pl.BlockSpec((1, 1, _FB), lambda e, f: (2 * k + e, 0, f)),
            pl.BlockSpec((1, _FB, D_MODEL), lambda e, f: (2 * k + e, f, 0)),
            pl.BlockSpec((1, 1, D_MODEL), lambda e, f: (2 * k + e, 0, 0)),
        ],
        out_specs=pl.BlockSpec((CP, D_MODEL), lambda e, f: (2 * k + e, 0)),
        out_shape=jax.ShapeDtypeStruct((YO_ROWS, D_MODEL), jnp.float32),
        input_output_aliases={0: 0},
        name=f"moe_ffn_{k}",
    )(yo_in, xi_q, W1, B1r, W2, B2r)


_O_HALF = TOKENS // 2
_O_CHUNK = _O_HALF // _NW


def _outgather_body(half, yo_hbm, inv_hbm, out_hbm, idx_v, rows_v, sem):
    wid = lax.axis_index("s") * _NC + lax.axis_index("c")
    off = wid * _O_CHUNK
    pltpu.sync_copy(inv_hbm.at[pl.ds(half * _O_HALF + off, _O_CHUNK)], idx_v)
    pltpu.async_copy(yo_hbm.at[idx_v], rows_v, sem).wait()
    pltpu.sync_copy(rows_v, out_hbm.at[pl.ds(off, _O_CHUNK)])


def _outgather_half(yo, inv, half):
    return pl.kernel(
        functools.partial(_outgather_body, half),
        out_type=jax.ShapeDtypeStruct((_O_HALF, D_MODEL), jnp.float32),
        mesh=_sc_mesh(),
        scratch_types=[
            pltpu.VMEM((_O_CHUNK,), jnp.int32),
            pltpu.VMEM((_O_CHUNK, D_MODEL), jnp.float32),
            pltpu.SemaphoreType.DMA,
        ],
        name=f"moe_outgather_{half}",
    )(yo, inv)


def kernel(x, Wr, br, W1, B1, W2, B2):
    B, S, D = x.shape
    xf = x.reshape(-1, D)
    wr_p = jnp.pad(Wr, ((0, 0), (0, 128 - N_EXP)))
    br_p = jnp.concatenate([br, jnp.full((128 - N_EXP,), -1e30, jnp.float32)])
    t12 = _router(xf, wr_p, br_p.reshape(1, 128))
    t12_t = t12.reshape(2, _L, _STRIPE).transpose(0, 2, 1).reshape(2, TOKENS)
    fwd, invt = _dispatch(t12_t)
    inv = invt.reshape(_STRIPE, _L).transpose(1, 0).reshape(TOKENS)
    B1r = B1.reshape(N_EXP, 1, D_FF)
    B2r = B2.reshape(N_EXP, 1, D_MODEL)
    yo = jnp.zeros((YO_ROWS, D_MODEL), jnp.float32)
    for k in range(4):
        xi_k = _gather_q(xf, fwd, k)
        yo = _ffn_q(k, yo, xi_k, W1, B1r, W2, B2r)
    o0 = _outgather_half(yo, inv, 0)
    o1 = _outgather_half(yo, inv, 1)
    out = jnp.concatenate([o0, o1], axis=0)
    return out.reshape(B, S, D)


# speedup vs baseline: 1.7751x; 1.0013x over previous
import functools
import math

import jax
import jax.numpy as jnp
from jax import lax
from jax.experimental import pallas as pl
from jax.experimental.pallas import tpu as pltpu
from jax.experimental.pallas import tpu_sc as plsc

D_MODEL = 1024
D_FF = 4096
N_EXP = 8
TOKENS = 4096
CAP = int(math.ceil(TOKENS * 1.6 / N_EXP))
CP = 896
NSLOT = N_EXP * CP
ZROW = NSLOT
YO_ROWS = (N_EXP + 1) * CP

_NC = 2
_NS = 16
_NW = _NC * _NS
_L = 16

_STRIPE = TOKENS // _L
_NITER = _STRIPE


def _sc_mesh():
    return plsc.VectorSubcoreMesh(
        core_axis_name="c", subcore_axis_name="s", num_cores=_NC, num_subcores=_NS
    )


def _router_body(x_ref, wr_ref, br_ref, t12_ref):
    l = jnp.dot(x_ref[...], wr_ref[...], preferred_element_type=jnp.float32)
    l = l + br_ref[...]
    col = lax.broadcasted_iota(jnp.int32, l.shape, 1)
    l = jnp.where(col < N_EXP, l, -jnp.inf)
    m1 = jnp.max(l, axis=1, keepdims=True)
    a1 = jnp.min(jnp.where(l == m1, col, 128), axis=1)
    l2 = jnp.where(col == a1[:, None], -jnp.inf, l)
    m2 = jnp.max(l2, axis=1, keepdims=True)
    a2 = jnp.min(jnp.where(l2 == m2, col, 128), axis=1)
    t12_ref[0:1, :] = a1.reshape(1, -1)
    t12_ref[1:2, :] = a2.reshape(1, -1)


def _router(xf, wr_p, br_p):
    return pl.pallas_call(
        _router_body,
        out_shape=jax.ShapeDtypeStruct((2, TOKENS), jnp.int32),
    )(xf, wr_p, br_p)


_DNUMS = lax.GatherDimensionNumbers(
    offset_dims=(), collapsed_slice_dims=(0,), start_index_map=(0,)
)


def _perm(x, idx):
    return lax.gather(
        x, idx[:, None], _DNUMS, (1,), mode=lax.GatherScatterMode.PROMISE_IN_BOUNDS
    )


def _excl_prefix(x):
    lane = lax.iota(jnp.int32, _L)
    s = x
    for k in (1, 2, 4, 8):
        sh = _perm(s, jnp.maximum(lane - k, 0))
        s = s + jnp.where(lane >= k, sh, 0)
    sh = _perm(s, jnp.maximum(lane - 1, 0))
    return jnp.where(lane >= 1, sh, 0)


def _dispatch_body(t12_hbm, fwd_hbm, invt_hbm,
                   t1_v, t2_v, wexp_v, invm_v, tokm_v, zb_v, cnt_v,
                   sh_v, inv2_v, tok2_v, sem):
    wid = lax.axis_index("s") * _NC + lax.axis_index("c")

    def _seg(k):
        return pl.ds(k * _L, _L)

    _ONE = jnp.full((_L,), 1, jnp.int32)
    _NIL = jnp.full((_L,), 0, jnp.int32)

    def _cnt(m):
        return jnp.where(m, _ONE, _NIL)

    @pl.when(wid == 0)
    def _():
        pltpu.sync_copy(t12_hbm.at[0], t1_v)
        pltpu.sync_copy(t12_hbm.at[1], t2_v)

        def czero(i, c):
            cnt_v[pl.ds(i * _L, _L)] = jnp.zeros((_L,), jnp.int32)
            return c

        lax.fori_loop(0, 48, czero, 0)

        def p1(i, c):
            acc = [_NIL] * N_EXP
            for u in range(8):
                t1 = t1_v[pl.ds((i * 8 + u) * _L, _L)]
                t2 = t2_v[pl.ds((i * 8 + u) * _L, _L)]
                for e in range(N_EXP):
                    m = (t1 == e) | (t2 == e)
                    acc[e] = acc[e] + _cnt(m)
            for e in range(N_EXP):
                cnt_v[_seg(16 + e)] = cnt_v[_seg(16 + e)] + acc[e]
            return c

        lax.fori_loop(0, _NITER // 8, p1, 0)
        for e in range(N_EXP):
            cnt_v[_seg(e)] = _excl_prefix(cnt_v[_seg(16 + e)])

        def p2(i, c):
            pos = [cnt_v[_seg(e)] + cnt_v[_seg(24 + e)] for e in range(N_EXP)]
            wacc = [_NIL] * N_EXP
            for u in range(4):
                t1 = t1_v[pl.ds((i * 4 + u) * _L, _L)]
                t2 = t2_v[pl.ds((i * 4 + u) * _L, _L)]
                wexp = jnp.full((_L,), -1, jnp.int32)
                for e in range(N_EXP):
                    m = (t1 == e) | (t2 == e)
                    kept = m & (pos[e] < CAP)
                    pos[e] = pos[e] + _cnt(m)
                    wexp = jnp.where(kept, jnp.full((_L,), e, jnp.int32), wexp)
                wexp_v[pl.ds((i * 4 + u) * _L, _L)] = wexp
                for e in range(N_EXP):
                    wacc[e] = wacc[e] + _cnt(wexp == e)
            for e in range(N_EXP):
                cnt_v[_seg(24 + e)] = pos[e] - cnt_v[_seg(e)]
                cnt_v[_seg(32 + e)] = cnt_v[_seg(32 + e)] + wacc[e]
            return c

        lax.fori_loop(0, _NITER // 4, p2, 0)
        for e in range(N_EXP):
            cnt_v[_seg(8 + e)] = _excl_prefix(cnt_v[_seg(32 + e)])

        def p3(i, c):
            slotb = [cnt_v[_seg(8 + e)] + cnt_v[_seg(40 + e)] + e * CP
                     for e in range(N_EXP)]
            for u in range(8):
                wexp = wexp_v[pl.ds((i * 8 + u) * _L, _L)]
                tok = lax.iota(jnp.int32, _L) * _STRIPE + i * 8 + u
                inv = ZROW + (tok & 511)
                for e in range(N_EXP):
                    mw = wexp == e
                    inv = jnp.where(mw, slotb[e], inv)
                    slotb[e] = slotb[e] + _cnt(mw)
                invm_v[i, pl.ds(u * _L, _L)] = inv
                tokm_v[i, pl.ds(u * _L, _L)] = tok
            for e in range(N_EXP):
                cnt_v[_seg(40 + e)] = slotb[e] - cnt_v[_seg(8 + e)] - e * CP
            return c

        lax.fori_loop(0, _NITER // 8, p3, 0)

        pltpu.sync_copy(invm_v, invt_hbm)
        pltpu.sync_copy(invm_v, sh_v.at[pl.ds(0, 32)])
        pltpu.sync_copy(tokm_v, sh_v.at[pl.ds(32, 32)])

    cid = lax.axis_index("c")
    sid = lax.axis_index("s")

    @pl.when(cid == 0)
    def _():
        base = sid * _ZCH

        def memset(i, c):
            zb_v[pl.ds(i * _L, _L)] = (
                (lax.iota(jnp.int32, _L) + base + i * _L) & (TOKENS - 1)
            )
            return c

        lax.fori_loop(0, _ZN // _L, memset, 0)

        @pl.when(sid < _NS - 1)
        def _():
            pltpu.sync_copy(zb_v.at[pl.ds(0, _ZCH)], fwd_hbm.at[pl.ds(base, _ZCH)])

        @pl.when(sid == _NS - 1)
        def _():
            pltpu.sync_copy(zb_v, fwd_hbm.at[pl.ds(base, _ZN)])

    plsc.subcore_barrier()

    @pl.when(cid == 0)
    def _():
        pltpu.sync_copy(sh_v.at[pl.ds(2 * sid, 2)], inv2_v)
        pltpu.sync_copy(sh_v.at[pl.ds(32 + 2 * sid, 2)], tok2_v)
        for j in range(2):
            pltpu.async_copy(tok2_v.at[j], fwd_hbm.at[inv2_v.at[j]], sem)
        for j in range(2):
            pltpu.make_async_copy(
                tok2_v.at[j], fwd_hbm.at[inv2_v.at[j]], sem
            ).wait()


_FWD_ROWS = NSLOT + 512
_ZCH = _FWD_ROWS // _NS // 8 * 8
_ZN = _FWD_ROWS - (_NS - 1) * _ZCH


def _dispatch(t12_t):
    return pl.kernel(
        _dispatch_body,
        out_type=(
            jax.ShapeDtypeStruct((_FWD_ROWS,), jnp.int32),
            jax.ShapeDtypeStruct((TOKENS // 128, 128), jnp.int32),
        ),
        mesh=_sc_mesh(),
        scratch_types=[
            pltpu.VMEM((TOKENS,), jnp.int32),
            pltpu.VMEM((TOKENS,), jnp.int32),
            pltpu.VMEM((TOKENS,), jnp.int32),
            pltpu.VMEM((TOKENS // 128, 128), jnp.int32),
            pltpu.VMEM((TOKENS // 128, 128), jnp.int32),
            pltpu.VMEM((_ZN,), jnp.int32),
            pltpu.VMEM((48 * _L,), jnp.int32),
            pltpu.VMEM_SHARED((64, 128), jnp.int32),
            pltpu.VMEM((2, 128), jnp.int32),
            pltpu.VMEM((2, 128), jnp.int32),
            pltpu.SemaphoreType.DMA,
        ],
    )(t12_t)


_G_Q = NSLOT // 4
_G_CHUNK = _G_Q // _NW


def _gather_body(q, x_hbm, fwd_hbm, xi_hbm, idx_v, rows_v, sem):
    wid = lax.axis_index("s") * _NC + lax.axis_index("c")
    off = wid * _G_CHUNK
    pltpu.sync_copy(fwd_hbm.at[pl.ds(q * _G_Q + off, _G_CHUNK)], idx_v)
    pltpu.async_copy(x_hbm.at[idx_v], rows_v, sem).wait()
    pltpu.sync_copy(rows_v, xi_hbm.at[pl.ds(off, _G_CHUNK)])


def _gather_q(xf, fwd, q):
    return pl.kernel(
        functools.partial(_gather_body, q),
        out_type=jax.ShapeDtypeStruct((_G_Q, D_MODEL), jnp.float32),
        mesh=_sc_mesh(),
        scratch_types=[
            pltpu.VMEM((_G_CHUNK,), jnp.int32),
            pltpu.VMEM((_G_CHUNK, D_MODEL), jnp.float32),
            pltpu.SemaphoreType.DMA,
        ],
        name=f"moe_gather_{q}",
    )(xf, fwd)


_FB = 512


def _ffn_body(yo_in_ref, xi_ref, w1_ref, b1_ref, w2_ref, b2_ref, out_ref):
    f = pl.program_id(1)
    xi = xi_ref[...].astype(jnp.bfloat16)
    w1 = w1_ref[0].astype(jnp.bfloat16)
    h = jnp.dot(xi, w1, preferred_element_type=jnp.float32) + b1_ref[0]
    h = 0.5 * h * (1.0 + lax.erf(h * (1.0 / math.sqrt(2.0))))
    w2 = w2_ref[0].astype(jnp.bfloat16)
    y = jnp.dot(h.astype(jnp.bfloat16), w2, preferred_element_type=jnp.float32)

    @pl.when(f == 0)
    def _():
        out_ref[...] = y + b2_ref[0]

    @pl.when(f > 0)
    def _():
        out_ref[...] = out_ref[...] + y


def _ffn_q(k, yo_in, xi_q, W1, B1r, W2, B2r):
    nf = D_FF // _FB
    return pl.pallas_call(
        _ffn_body,
        grid=(2, nf),
        in_specs=[
            pl.BlockSpec(memory_space=pl.ANY),
            pl.BlockSpec((CP, D_MODEL), lambda e, f: (e, 0)),
            pl.BlockSpec((1, D_MODEL, _FB), lambda e, f: (2 * k + e, 0, f)),
            pl.BlockSpec((1, 1, _FB), lambda e, f: (2 * k + e, 0, f)),
            pl.BlockSpec((1, _FB, D_MODEL), lambda e, f: (2 * k + e, f, 0)),
            pl.BlockSpec((1, 1, D_MODEL), lambda e, f: (2 * k + e, 0, 0)),
        ],
        out_specs=pl.BlockSpec((CP, D_MODEL), lambda e, f: (2 * k + e, 0)),
        out_shape=jax.ShapeDtypeStruct((YO_ROWS, D_MODEL), jnp.float32),
        input_output_aliases={0: 0},
        name=f"moe_ffn_{k}",
    )(yo_in, xi_q, W1, B1r, W2, B2r)


_O_HALF = TOKENS // 2
_O_CHUNK = _O_HALF // _NW


def _outgather_body(half, yo_hbm, inv_hbm, out_hbm, idx_v, rows_v, sem):
    wid = lax.axis_index("s") * _NC + lax.axis_index("c")
    off = wid * _O_CHUNK
    pltpu.sync_copy(inv_hbm.at[pl.ds(half * _O_HALF + off, _O_CHUNK)], idx_v)
    pltpu.async_copy(yo_hbm.at[idx_v], rows_v, sem).wait()
    pltpu.sync_copy(rows_v, out_hbm.at[pl.ds(off, _O_CHUNK)])


def _outgather_half(yo, inv, half):
    return pl.kernel(
        functools.partial(_outgather_body, half),
        out_type=jax.ShapeDtypeStruct((_O_HALF, D_MODEL), jnp.float32),
        mesh=_sc_mesh(),
        scratch_types=[
            pltpu.VMEM((_O_CHUNK,), jnp.int32),
            pltpu.VMEM((_O_CHUNK, D_MODEL), jnp.float32),
            pltpu.SemaphoreType.DMA,
        ],
        name=f"moe_outgather_{half}",
    )(yo, inv)


def kernel(x, Wr, br, W1, B1, W2, B2):
    B, S, D = x.shape
    xf = x.reshape(-1, D)
    t12 = _router(xf, Wr, br.reshape(1, N_EXP))
    t12_t = t12.reshape(2, _L, _STRIPE).transpose(0, 2, 1).reshape(2, TOKENS)
    fwd, invt = _dispatch(t12_t)
    inv = invt.reshape(_STRIPE, _L).transpose(1, 0).reshape(TOKENS)
    B1r = B1.reshape(N_EXP, 1, D_FF)
    B2r = B2.reshape(N_EXP, 1, D_MODEL)
    yo = jnp.zeros((YO_ROWS, D_MODEL), jnp.float32)
    for k in range(4):
        xi_k = _gather_q(xf, fwd, k)
        yo = _ffn_q(k, yo, xi_k, W1, B1r, W2, B2r)
    o0 = _outgather_half(yo, inv, 0)
    o1 = _outgather_half(yo, inv, 1)
    out = jnp.concatenate([o0, o1], axis=0)
    return out.reshape(B, S, D)


# speedup vs baseline: 1.8300x; 1.0309x over previous
import functools
import math

import jax
import jax.numpy as jnp
from jax import lax
from jax.experimental import pallas as pl
from jax.experimental.pallas import tpu as pltpu
from jax.experimental.pallas import tpu_sc as plsc

D_MODEL = 1024
D_FF = 4096
N_EXP = 8
TOKENS = 4096
CAP = int(math.ceil(TOKENS * 1.6 / N_EXP))
CP = 896
NSLOT = N_EXP * CP
ZROW = NSLOT
YO_ROWS = (N_EXP + 1) * CP

_NC = 2
_NS = 16
_NW = _NC * _NS
_L = 16

_STRIPE = TOKENS // _L
_NITER = _STRIPE


def _sc_mesh():
    return plsc.VectorSubcoreMesh(
        core_axis_name="c", subcore_axis_name="s", num_cores=_NC, num_subcores=_NS
    )


def _router_body(x_ref, wr_ref, br_ref, t12_ref):
    l = jnp.dot(x_ref[...], wr_ref[...], preferred_element_type=jnp.float32)
    l = l + br_ref[...]
    col = lax.broadcasted_iota(jnp.int32, l.shape, 1)
    l = jnp.where(col < N_EXP, l, -jnp.inf)
    m1 = jnp.max(l, axis=1, keepdims=True)
    a1 = jnp.min(jnp.where(l == m1, col, 128), axis=1)
    l2 = jnp.where(col == a1[:, None], -jnp.inf, l)
    m2 = jnp.max(l2, axis=1, keepdims=True)
    a2 = jnp.min(jnp.where(l2 == m2, col, 128), axis=1)
    t12_ref[0:1, :] = a1.reshape(1, -1)
    t12_ref[1:2, :] = a2.reshape(1, -1)


def _router(xf, wr_p, br_p):
    return pl.pallas_call(
        _router_body,
        out_shape=jax.ShapeDtypeStruct((2, TOKENS), jnp.int32),
    )(xf, wr_p, br_p)


_DNUMS = lax.GatherDimensionNumbers(
    offset_dims=(), collapsed_slice_dims=(0,), start_index_map=(0,)
)


def _perm(x, idx):
    return lax.gather(
        x, idx[:, None], _DNUMS, (1,), mode=lax.GatherScatterMode.PROMISE_IN_BOUNDS
    )


def _excl_prefix(x):
    lane = lax.iota(jnp.int32, _L)
    s = x
    for k in (1, 2, 4, 8):
        sh = _perm(s, jnp.maximum(lane - k, 0))
        s = s + jnp.where(lane >= k, sh, 0)
    sh = _perm(s, jnp.maximum(lane - 1, 0))
    return jnp.where(lane >= 1, sh, 0)


def _dispatch_body(t12_hbm, fwd_hbm, invt_hbm,
                   t1_v, t2_v, wexp_v, invm_v, tokm_v, zb_v, cnt_v,
                   sh_v, stg_v, loc_v, locb_v, sem):
    cid = lax.axis_index("c")
    sid = lax.axis_index("s")
    lane = lax.iota(jnp.int32, _L)

    def _seg(k):
        return pl.ds(k * _L, _L)

    _ONE = jnp.full((_L,), 1, jnp.int32)
    _NIL = jnp.full((_L,), 0, jnp.int32)

    def _cnt(m):
        return jnp.where(m, _ONE, _NIL)

    def _incl_prefix(x):
        t = x
        for k in (1, 2, 4, 8):
            sh = _perm(t, jnp.maximum(lane - k, 0))
            t = t + jnp.where(lane >= k, sh, 0)
        return t

    def _shift1(x):
        sh = _perm(x, jnp.maximum(lane - 1, 0))
        return jnp.where(lane >= 1, sh, 0)

    @pl.when(cid == 0)
    def _():
        pltpu.sync_copy(t12_hbm.at[pl.ds(sid * _TPW, _TPW)], t1_v)
        pltpu.sync_copy(t12_hbm.at[pl.ds(TOKENS + sid * _TPW, _TPW)], t2_v)
        base = sid * _ZCH

        def memset(i, c):
            zb_v[pl.ds(i * _L, _L)] = (
                (lax.iota(jnp.int32, _L) + base + i * _L) & (TOKENS - 1)
            )
            return c

        lax.fori_loop(0, _ZN // _L, memset, 0)

        @pl.when(sid < _NS - 1)
        def _():
            pltpu.sync_copy(zb_v.at[pl.ds(0, _ZCH)], fwd_hbm.at[pl.ds(base, _ZCH)])

        @pl.when(sid == _NS - 1)
        def _():
            pltpu.sync_copy(zb_v, fwd_hbm.at[pl.ds(base, _ZN)])

        def czero(i, c):
            cnt_v[pl.ds((16 + i) * _L, _L)] = jnp.zeros((_L,), jnp.int32)
            return c

        lax.fori_loop(0, 32, czero, 0)

        def p1(i, c):
            acc = [_NIL] * N_EXP
            for u in range(8):
                t1 = t1_v[pl.ds((i * 8 + u) * _L, _L)]
                t2 = t2_v[pl.ds((i * 8 + u) * _L, _L)]
                for e in range(N_EXP):
                    m = (t1 == e) | (t2 == e)
                    acc[e] = acc[e] + _cnt(m)
            for e in range(N_EXP):
                cnt_v[_seg(16 + e)] = cnt_v[_seg(16 + e)] + acc[e]
            return c

        lax.fori_loop(0, _TPW // _L // 8, p1, 0)
        comb = _NIL
        for e in range(N_EXP):
            sinc = _incl_prefix(cnt_v[_seg(16 + e)])
            cnt_v[_seg(e)] = _shift1(sinc)
            tot = _perm(sinc, jnp.full((_L,), _L - 1, jnp.int32))
            comb = jnp.where(lane == e, tot, comb)
        stg_v[...] = comb
        pltpu.sync_copy(stg_v, sh_v.at[pl.ds(sid * _L, _L)])

    plsc.subcore_barrier()

    @pl.when((cid == 0) & (sid == 0))
    def _():
        pltpu.sync_copy(sh_v.at[pl.ds(0, _NS * _L)], loc_v)
        acc = _NIL
        for r in range(_NS):
            locb_v[pl.ds(r * _L, _L)] = acc
            acc = acc + loc_v[pl.ds(r * _L, _L)]
        pltpu.sync_copy(locb_v, sh_v.at[pl.ds(_NS * _L, _NS * _L)])

    plsc.subcore_barrier()

    @pl.when(cid == 0)
    def _():
        pltpu.sync_copy(sh_v.at[pl.ds(_NS * _L + sid * _L, _L)], stg_v)
        brow = stg_v[...]
        for e in range(N_EXP):
            tb = _perm(brow, jnp.full((_L,), e, jnp.int32))
            cnt_v[_seg(e)] = cnt_v[_seg(e)] + tb

        def p2(i, c):
            pos = [cnt_v[_seg(e)] + cnt_v[_seg(24 + e)] for e in range(N_EXP)]
            wacc = [_NIL] * N_EXP
            for u in range(4):
                t1 = t1_v[pl.ds((i * 4 + u) * _L, _L)]
                t2 = t2_v[pl.ds((i * 4 + u) * _L, _L)]
                wexp = jnp.full((_L,), -1, jnp.int32)
                for e in range(N_EXP):
                    m = (t1 == e) | (t2 == e)
                    kept = m & (pos[e] < CAP)
                    pos[e] = pos[e] + _cnt(m)
                    wexp = jnp.where(kept, jnp.full((_L,), e, jnp.int32), wexp)
                wexp_v[pl.ds((i * 4 + u) * _L, _L)] = wexp
                for e in range(N_EXP):
                    wacc[e] = wacc[e] + _cnt(wexp == e)
            for e in range(N_EXP):
                cnt_v[_seg(24 + e)] = pos[e] - cnt_v[_seg(e)]
                cnt_v[_seg(32 + e)] = cnt_v[_seg(32 + e)] + wacc[e]
            return c

        lax.fori_loop(0, _TPW // _L // 4, p2, 0)
        comb = _NIL
        for e in range(N_EXP):
            sinc = _incl_prefix(cnt_v[_seg(32 + e)])
            cnt_v[_seg(8 + e)] = _shift1(sinc)
            tot = _perm(sinc, jnp.full((_L,), _L - 1, jnp.int32))
            comb = jnp.where(lane == e, tot, comb)
        stg_v[...] = comb
        pltpu.sync_copy(stg_v, sh_v.at[pl.ds(2 * _NS * _L + sid * _L, _L)])

    plsc.subcore_barrier()

    @pl.when((cid == 0) & (sid == 0))
    def _():
        pltpu.sync_copy(sh_v.at[pl.ds(2 * _NS * _L, _NS * _L)], loc_v)
        acc = _NIL
        for r in range(_NS):
            locb_v[pl.ds(r * _L, _L)] = acc
            acc = acc + loc_v[pl.ds(r * _L, _L)]
        pltpu.sync_copy(locb_v, sh_v.at[pl.ds(3 * _NS * _L, _NS * _L)])

    plsc.subcore_barrier()

    @pl.when(cid == 0)
    def _():
        pltpu.sync_copy(sh_v.at[pl.ds(3 * _NS * _L + sid * _L, _L)], stg_v)
        brow = stg_v[...]
        for e in range(N_EXP):
            tb = _perm(brow, jnp.full((_L,), e, jnp.int32))
            cnt_v[_seg(8 + e)] = cnt_v[_seg(8 + e)] + tb

        def p3(i, c):
            slotb = [cnt_v[_seg(8 + e)] + cnt_v[_seg(40 + e)] + e * CP
                     for e in range(N_EXP)]
            for u in range(8):
                wexp = wexp_v[pl.ds((i * 8 + u) * _L, _L)]
                tok = sid * _TPW + lane * _L + i * 8 + u
                inv = ZROW + (tok & 511)
                for e in range(N_EXP):
                    mw = wexp == e
                    inv = jnp.where(mw, slotb[e], inv)
                    slotb[e] = slotb[e] + _cnt(mw)
                invm_v[i, pl.ds(u * _L, _L)] = inv
                tokm_v[i, pl.ds(u * _L, _L)] = tok
            for e in range(N_EXP):
                cnt_v[_seg(40 + e)] = slotb[e] - cnt_v[_seg(8 + e)] - e * CP
            return c

        lax.fori_loop(0, _TPW // _L // 8, p3, 0)
        pltpu.sync_copy(invm_v, invt_hbm.at[pl.ds(2 * sid, 2)])
        for j in range(2):
            pltpu.async_copy(tokm_v.at[j], fwd_hbm.at[invm_v.at[j]], sem)
        for j in range(2):
            pltpu.make_async_copy(
                tokm_v.at[j], fwd_hbm.at[invm_v.at[j]], sem
            ).wait()


_TPW = TOKENS // _NS
_FWD_ROWS = NSLOT + 512
_ZCH = _FWD_ROWS // _NS // 8 * 8
_ZN = _FWD_ROWS - (_NS - 1) * _ZCH


def _dispatch(t12_t):
    return pl.kernel(
        _dispatch_body,
        out_type=(
            jax.ShapeDtypeStruct((_FWD_ROWS,), jnp.int32),
            jax.ShapeDtypeStruct((TOKENS // 128, 128), jnp.int32),
        ),
        mesh=_sc_mesh(),
        scratch_types=[
            pltpu.VMEM((_TPW,), jnp.int32),
            pltpu.VMEM((_TPW,), jnp.int32),
            pltpu.VMEM((_TPW,), jnp.int32),
            pltpu.VMEM((2, 128), jnp.int32),
            pltpu.VMEM((2, 128), jnp.int32),
            pltpu.VMEM((_ZN,), jnp.int32),
            pltpu.VMEM((48 * _L,), jnp.int32),
            pltpu.VMEM_SHARED((4 * _NS * _L,), jnp.int32),
            pltpu.VMEM((_L,), jnp.int32),
            pltpu.VMEM((_NS * _L,), jnp.int32),
            pltpu.VMEM((_NS * _L,), jnp.int32),
            pltpu.SemaphoreType.DMA,
        ],
    )(t12_t)


_G_Q = NSLOT // 4
_G_CHUNK = _G_Q // _NW


def _gather_body(q, x_hbm, fwd_hbm, xi_hbm, idx_v, rows_v, sem):
    wid = lax.axis_index("s") * _NC + lax.axis_index("c")
    off = wid * _G_CHUNK
    pltpu.sync_copy(fwd_hbm.at[pl.ds(q * _G_Q + off, _G_CHUNK)], idx_v)
    pltpu.async_copy(x_hbm.at[idx_v], rows_v, sem).wait()
    pltpu.sync_copy(rows_v, xi_hbm.at[pl.ds(off, _G_CHUNK)])


def _gather_q(xf, fwd, q):
    return pl.kernel(
        functools.partial(_gather_body, q),
        out_type=jax.ShapeDtypeStruct((_G_Q, D_MODEL), jnp.float32),
        mesh=_sc_mesh(),
        scratch_types=[
            pltpu.VMEM((_G_CHUNK,), jnp.int32),
            pltpu.VMEM((_G_CHUNK, D_MODEL), jnp.float32),
            pltpu.SemaphoreType.DMA,
        ],
        name=f"moe_gather_{q}",
    )(xf, fwd)


_FB = 512


def _ffn_body(yo_in_ref, xi_ref, w1_ref, b1_ref, w2_ref, b2_ref, out_ref):
    f = pl.program_id(1)
    xi = xi_ref[...].astype(jnp.bfloat16)
    w1 = w1_ref[0].astype(jnp.bfloat16)
    h = jnp.dot(xi, w1, preferred_element_type=jnp.float32) + b1_ref[0]
    h = 0.5 * h * (1.0 + lax.erf(h * (1.0 / math.sqrt(2.0))))
    w2 = w2_ref[0].astype(jnp.bfloat16)
    y = jnp.dot(h.astype(jnp.bfloat16), w2, preferred_element_type=jnp.float32)

    @pl.when(f == 0)
    def _():
        out_ref[...] = y + b2_ref[0]

    @pl.when(f > 0)
    def _():
        out_ref[...] = out_ref[...] + y


def _ffn_q(k, yo_in, xi_q, W1, B1r, W2, B2r):
    nf = D_FF // _FB
    return pl.pallas_call(
        _ffn_body,
        grid=(2, nf),
        in_specs=[
            pl.BlockSpec(memory_space=pl.ANY),
            pl.BlockSpec((CP, D_MODEL), lambda e, f: (e, 0)),
            pl.BlockSpec((1, D_MODEL, _FB), lambda e, f: (2 * k + e, 0, f)),
            pl.BlockSpec((1, 1, _FB), lambda e, f: (2 * k + e, 0, f)),
            pl.BlockSpec((1, _FB, D_MODEL), lambda e, f: (2 * k + e, f, 0)),
            pl.BlockSpec((1, 1, D_MODEL), lambda e, f: (2 * k + e, 0, 0)),
        ],
        out_specs=pl.BlockSpec((CP, D_MODEL), lambda e, f: (2 * k + e, 0)),
        out_shape=jax.ShapeDtypeStruct((YO_ROWS, D_MODEL), jnp.float32),
        input_output_aliases={0: 0},
        name=f"moe_ffn_{k}",
    )(yo_in, xi_q, W1, B1r, W2, B2r)


_O_HALF = TOKENS // 2
_O_CHUNK = _O_HALF // _NW


def _outgather_body(half, yo_hbm, inv_hbm, out_hbm, idx_v, rows_v, sem):
    wid = lax.axis_index("s") * _NC + lax.axis_index("c")
    off = wid * _O_CHUNK
    pltpu.sync_copy(inv_hbm.at[pl.ds(half * _O_HALF + off, _O_CHUNK)], idx_v)
    pltpu.async_copy(yo_hbm.at[idx_v], rows_v, sem).wait()
    pltpu.sync_copy(rows_v, out_hbm.at[pl.ds(off, _O_CHUNK)])


def _outgather_half(yo, inv, half):
    return pl.kernel(
        functools.partial(_outgather_body, half),
        out_type=jax.ShapeDtypeStruct((_O_HALF, D_MODEL), jnp.float32),
        mesh=_sc_mesh(),
        scratch_types=[
            pltpu.VMEM((_O_CHUNK,), jnp.int32),
            pltpu.VMEM((_O_CHUNK, D_MODEL), jnp.float32),
            pltpu.SemaphoreType.DMA,
        ],
        name=f"moe_outgather_{half}",
    )(yo, inv)


def kernel(x, Wr, br, W1, B1, W2, B2):
    B, S, D = x.shape
    xf = x.reshape(-1, D)
    t12 = _router(xf, Wr, br.reshape(1, N_EXP))
    t12_t = (
        t12.reshape(2, _NS, _L, _L).transpose(0, 1, 3, 2).reshape(2 * TOKENS)
    )
    fwd, invt = _dispatch(t12_t)
    inv = invt.reshape(_NS, _L, _L).transpose(0, 2, 1).reshape(TOKENS)
    B1r = B1.reshape(N_EXP, 1, D_FF)
    B2r = B2.reshape(N_EXP, 1, D_MODEL)
    yo = jnp.zeros((YO_ROWS, D_MODEL), jnp.float32)
    for k in range(4):
        xi_k = _gather_q(xf, fwd, k)
        yo = _ffn_q(k, yo, xi_k, W1, B1r, W2, B2r)
    o0 = _outgather_half(yo, inv, 0)
    o1 = _outgather_half(yo, inv, 1)
    out = jnp.concatenate([o0, o1], axis=0)
    return out.reshape(B, S, D)
